# Initial kernel scaffold; baseline (speedup 1.0000x reference)
import jax
import jax.numpy as jnp
from jax.experimental import pallas as pl


def kernel(x_cat, x_cont, emb_tables, bn_cont_g, bn_cont_b, W1, b1, g1, be1, W2, b2, g2, be2, W3, b3, g3, be3, W_out, b_out):
    raise NotImplementedError("write your pallas kernel here")


# speedup vs baseline: 15.3234x; 15.3234x over previous
import functools

import jax
import jax.numpy as jnp
from jax import lax
from jax.experimental import pallas as pl
from jax.experimental.pallas import tpu as pltpu
from jax.experimental.pallas import tpu_sc as plsc

_B = 4096
_NF = 26
_V = 1000
_D = 128
_KE = _NF * _D
_NC = 13
_EPS = 1e-5


_NW = 32
_ROWS = _B * _NF
_RPW = _ROWS // _NW
_CH = 128
_NCHUNK = _RPW // _CH
_DW = _D // 2


_CPF = _B // _CH
_BH = _B // 2
_CPFH = _BH // _CH
_NCH_H = _NF * _CPFH // _NW


def _sc_gather_kernel(table_hbm, idx_hbm, out_hbm, idx_v, buf0, buf1, sem0, sem1):
    wid = lax.axis_index("s") * 2 + lax.axis_index("c")
    chunk_base = wid * _NCH_H
    pltpu.sync_copy(idx_hbm.at[wid], idx_v)

    def dst(c):
        f = c // _CPFH
        b0 = pl.multiple_of((c % _CPFH) * _CH, _CH)
        col = pl.multiple_of(f * _D, _D)
        return out_hbm.at[pl.ds(b0, _CH), pl.ds(col, _D)]

    def body(i, carry):
        j0 = 2 * i
        j1 = j0 + 1
        g0 = pltpu.async_copy(table_hbm.at[idx_v.at[j0]], buf0, sem0)
        g1 = pltpu.async_copy(table_hbm.at[idx_v.at[j1]], buf1, sem1)
        g0.wait()
        pltpu.sync_copy(buf0, dst(chunk_base + j0))
        g1.wait()
        pltpu.sync_copy(buf1, dst(chunk_base + j1))
        return carry

    lax.fori_loop(0, _NCH_H // 2, body, 0)
    jt = _NCH_H - 1
    gt = pltpu.async_copy(table_hbm.at[idx_v.at[jt]], buf0, sem0)
    gt.wait()
    pltpu.sync_copy(buf0, dst(chunk_base + jt))


def _sc_gather(table, idx3d):
    mesh = plsc.VectorSubcoreMesh(core_axis_name="c", subcore_axis_name="s")
    f = functools.partial(
        pl.kernel,
        mesh=mesh,
        out_type=jax.ShapeDtypeStruct((_BH, _KE), jnp.float32),
        scratch_types=[
            pltpu.VMEM((_NCH_H, _CH), jnp.int32),
            pltpu.VMEM((_CH, _D), jnp.float32),
            pltpu.VMEM((_CH, _D), jnp.float32),
            pltpu.SemaphoreType.DMA,
            pltpu.SemaphoreType.DMA,
        ],
    )(_sc_gather_kernel)
    return f(table, idx3d)


_BM = 512
_NT = _B // _BM
_NTH = _BH // _BM
_L1, _L2, _L3 = 1024, 512, 256


def _layer1_body(base, emb_ref, xc_ref, gc_ref, bec_ref, w1_ref, b1_ref,
                 z_ref, ssum_ref, ssq_ref):
    j = pl.program_id(0)

    xc_all = xc_ref[...]
    m = jnp.mean(xc_all, axis=0, keepdims=True)
    v = jnp.mean(xc_all * xc_all, axis=0, keepdims=True) - m * m
    a = lax.rsqrt(v + _EPS)
    xc_n = (((xc_ref[pl.ds(base + j * _BM, _BM), :] - m) * a) * gc_ref[...]
            + bec_ref[...]).astype(jnp.bfloat16)

    z = lax.dot_general(emb_ref[...].astype(jnp.bfloat16), w1_ref[:, :_KE],
                        (((1,), (1,)), ((), ())),
                        preferred_element_type=jnp.float32)
    z += lax.dot_general(xc_n, w1_ref[:, _KE:], (((1,), (1,)), ((), ())),
                         preferred_element_type=jnp.float32)
    z = jnp.maximum(z + b1_ref[...], 0.0)
    z_ref[...] = z.astype(jnp.bfloat16)

    @pl.when(j == 0)
    def _():
        ssum_ref[...] = jnp.zeros_like(ssum_ref)
        ssq_ref[...] = jnp.zeros_like(ssq_ref)

    ssum_ref[...] += jnp.sum(z, axis=0, keepdims=True)
    ssq_ref[...] += jnp.sum(z * z, axis=0, keepdims=True)


def _layer1_half(emb_half, x_cont, gc, bec, w1, b1, base):
    const = lambda s: pl.BlockSpec(s, lambda j: (0, 0))
    return pl.pallas_call(
        functools.partial(_layer1_body, base),
        grid=(_NTH,),
        in_specs=[
            pl.BlockSpec((_BM, _KE), lambda j: (j, 0)),
            const((_B, _NC)), const((1, _NC)), const((1, _NC)),
            const((_L1, _KE + _NC)), const((1, _L1)),
        ],
        out_specs=[
            pl.BlockSpec((_BM, _L1), lambda j: (j, 0)),
            const((1, _L1)), const((1, _L1)),
        ],
        out_shape=[
            jax.ShapeDtypeStruct((_BH, _L1), jnp.bfloat16),
            jax.ShapeDtypeStruct((1, _L1), jnp.float32),
            jax.ShapeDtypeStruct((1, _L1), jnp.float32),
        ],
    )(emb_half, x_cont, gc, bec, w1, b1)


def _bn_affine(ssum, ssq, g_ref, be_ref):
    m = ssum * (1.0 / _B)
    v = ssq * (1.0 / _B) - m * m
    a = g_ref[...] * lax.rsqrt(v + _EPS)
    return a, be_ref[...] - m * a


_BM2 = 1024


def _tail_body(emb_ref, z1a_ref, xc_ref, gc_ref, bec_ref, w1_ref, b1_ref,
               s1a_ref, q1a_ref,
               g1_ref, be1_ref, w2_ref, b2_ref,
               g2_ref, be2_ref, w3_ref, b3_ref,
               g3_ref, be3_ref, wo_ref, bo_ref,
               o_ref,
               z1b_s, z2_s, z3_s, s1b, q1b, s2, q2, s3, q3):
    ph = pl.program_id(0)
    j = pl.program_id(1)
    rows = pl.ds(j * _BM, _BM)

    @pl.when(ph == 0)
    def _():
        @pl.when(j == 0)
        def _():
            s1b[...] = jnp.zeros_like(s1b)
            q1b[...] = jnp.zeros_like(q1b)

        xc_all = xc_ref[...]
        m = jnp.mean(xc_all, axis=0, keepdims=True)
        v = jnp.mean(xc_all * xc_all, axis=0, keepdims=True) - m * m
        a = lax.rsqrt(v + _EPS)
        xc_n = (((xc_ref[pl.ds(_BH + j * _BM, _BM), :] - m) * a) * gc_ref[...]
                + bec_ref[...]).astype(jnp.bfloat16)
        z = lax.dot_general(emb_ref[...].astype(jnp.bfloat16), w1_ref[:, :_KE],
                            (((1,), (1,)), ((), ())),
                            preferred_element_type=jnp.float32)
        z += lax.dot_general(xc_n, w1_ref[:, _KE:], (((1,), (1,)), ((), ())),
                             preferred_element_type=jnp.float32)
        z = jnp.maximum(z + b1_ref[...], 0.0)
        z1b_s[rows, :] = z
        s1b[...] += jnp.sum(z, axis=0, keepdims=True)
        q1b[...] += jnp.sum(z * z, axis=0, keepdims=True)

    @pl.when(ph == 1)
    def _():
        @pl.when(j == 0)
        def _():
            s2[...] = jnp.zeros_like(s2)
            q2[...] = jnp.zeros_like(q2)

        a, c = _bn_affine(s1a_ref[...] + s1b[...], q1a_ref[...] + q1b[...],
                          g1_ref, be1_ref)
        xn = (z1a_ref[...].astype(jnp.float32) * a + c).astype(jnp.bfloat16)
        z = lax.dot_general(xn, w2_ref[...], (((1,), (1,)), ((), ())),
                            preferred_element_type=jnp.float32)
        z = jnp.maximum(z + b2_ref[...], 0.0)
        z2_s[rows, :] = z
        s2[...] += jnp.sum(z, axis=0, keepdims=True)
        q2[...] += jnp.sum(z * z, axis=0, keepdims=True)

    @pl.when(ph == 2)
    def _():
        a, c = _bn_affine(s1a_ref[...] + s1b[...], q1a_ref[...] + q1b[...],
                          g1_ref, be1_ref)
        xn = (z1b_s[rows, :] * a + c).astype(jnp.bfloat16)
        z = lax.dot_general(xn, w2_ref[...], (((1,), (1,)), ((), ())),
                            preferred_element_type=jnp.float32)
        z = jnp.maximum(z + b2_ref[...], 0.0)
        z2_s[pl.ds(_BH + j * _BM, _BM), :] = z
        s2[...] += jnp.sum(z, axis=0, keepdims=True)
        q2[...] += jnp.sum(z * z, axis=0, keepdims=True)

    rows2 = pl.ds(j * _BM2, _BM2)

    @pl.when(ph == 3)
    def _():
        @pl.when(j == 0)
        def _():
            s3[...] = jnp.zeros_like(s3)
            q3[...] = jnp.zeros_like(q3)

        a, c = _bn_affine(s2[...], q2[...], g2_ref, be2_ref)
        xn = (z2_s[rows2, :] * a + c).astype(jnp.bfloat16)
        z = lax.dot_general(xn, w3_ref[...], (((1,), (1,)), ((), ())),
                            preferred_element_type=jnp.float32)
        z = jnp.maximum(z + b3_ref[...], 0.0)
        z3_s[rows2, :] = z
        s3[...] += jnp.sum(z, axis=0, keepdims=True)
        q3[...] += jnp.sum(z * z, axis=0, keepdims=True)

    @pl.when(ph == 4)
    def _():
        a, c = _bn_affine(s3[...], q3[...], g3_ref, be3_ref)
        xn = z3_s[rows2, :] * a + c
        o = jnp.sum(xn * wo_ref[...], axis=1, keepdims=True)
        o_ref[...] = o + bo_ref[0, 0]


def _tail(emb_b, z1a, x_cont, gc, bec, w1, b1, s1a, q1a,
          g1, be1, w2, b2, g2, be2, w3, b3, g3, be3, wo, bo):
    const = lambda s: pl.BlockSpec(s, lambda i, j: (0, 0))
    return pl.pallas_call(
        _tail_body,
        grid=(5, _NTH),
        in_specs=[
            pl.BlockSpec((_BM, _KE),
                         lambda i, j: (lax.select(i == 0, j, 0), 0)),
            pl.BlockSpec((_BM, _L1),
                         lambda i, j: (lax.select(i == 1, j, 0), 0)),
            const((_B, _NC)), const((1, _NC)), const((1, _NC)),
            const((_L1, _KE + _NC)), const((1, _L1)),
            const((1, _L1)), const((1, _L1)),
            const((1, _L1)), const((1, _L1)),
            const((_L2, _L1)), const((1, _L2)),
            const((1, _L2)), const((1, _L2)),
            const((_L3, _L2)), const((1, _L3)),
            const((1, _L3)), const((1, _L3)),
            const((1, _L3)), const((1, 1)),
        ],
        out_specs=pl.BlockSpec((_BM2, 1),
                               lambda i, j: (lax.select(i == 4, j, 0), 0)),
        out_shape=jax.ShapeDtypeStruct((_B, 1), jnp.float32),
        scratch_shapes=[
            pltpu.VMEM((_BH, _L1), jnp.float32),
            pltpu.VMEM((_B, _L2), jnp.float32),
            pltpu.VMEM((_B, _L3), jnp.float32),
            pltpu.VMEM((1, _L1), jnp.float32),
            pltpu.VMEM((1, _L1), jnp.float32),
            pltpu.VMEM((1, _L2), jnp.float32),
            pltpu.VMEM((1, _L2), jnp.float32),
            pltpu.VMEM((1, _L3), jnp.float32),
            pltpu.VMEM((1, _L3), jnp.float32),
        ],
    )(emb_b, z1a, x_cont, gc, bec, w1, b1, s1a, q1a,
      g1, be1, w2, b2, g2, be2, w3, b3, g3, be3, wo, bo)


def kernel(x_cat, x_cont, emb_tables, bn_cont_g, bn_cont_b,
           W1, b1, g1, be1, W2, b2, g2, be2, W3, b3, g3, be3,
           W_out, b_out):
    table = emb_tables.reshape(_NF * _V, _D)
    xT = x_cat.astype(jnp.int32).T
    off = (jnp.arange(_NF, dtype=jnp.int32) * _V)[:, None]
    idx_a = (xT[:, :_BH] + off).reshape(_NW, _NCH_H, _CH)
    idx_b = (xT[:, _BH:] + off).reshape(_NW, _NCH_H, _CH)
    W1bf = W1.astype(jnp.bfloat16)
    gc, bec = bn_cont_g[None, :], bn_cont_b[None, :]

    emb_a = _sc_gather(table, idx_a)
    emb_b = _sc_gather(table, idx_b)
    z1a, s1a, q1a = _layer1_half(emb_a, x_cont, gc, bec, W1bf, b1[None, :], 0)

    return _tail(emb_b, z1a, x_cont, gc, bec, W1bf, b1[None, :], s1a, q1a,
                 g1[None, :], be1[None, :],
                 W2.astype(jnp.bfloat16), b2[None, :],
                 g2[None, :], be2[None, :],
                 W3.astype(jnp.bfloat16), b3[None, :],
                 g3[None, :], be3[None, :],
                 W_out, b_out[None, :])
